# baseline (speedup 1.0000x reference)
import jax
import jax.numpy as jnp
from jax.experimental import pallas as pl
from jax.experimental.pallas import tpu as pltpu

N = 4096
D = 1024
K = 5
BLOCK = 512
NBLK = N // BLOCK


def _loss_block_kernel(full_ref, lab_row_ref, lab_col_ref, out_ref,
                       rncol_ref):
    i = pl.program_id(0)
    lab_mine = lab_col_ref[...]

    @pl.when(i == 0)
    def _setup():
        full = full_ref[...]
        sq = full * full
        ones_row = jnp.ones((1, D), dtype=jnp.float32)
        col_ss = jax.lax.dot_general(ones_row, sq, (((1,), (1,)), ((), ())),
                                     preferred_element_type=jnp.float32)
        n_col = jnp.maximum(jnp.sqrt(col_ss), 1e-12)
        rncol_ref[...] = 1.0 / n_col
        out_ref[...] = jnp.zeros((1, 1), jnp.float32)

    rows = full_ref[pl.ds(i * BLOCK, BLOCK), :]
    row_ss = jnp.sum(rows * rows, axis=1, keepdims=True)
    n_rows = jnp.maximum(jnp.sqrt(row_ss), 1e-12)
    rows_s = rows * (1.0 / n_rows)

    p = jax.lax.dot_general(rows_s, full_ref[...], (((1,), (1,)), ((), ())),
                            preferred_element_type=jnp.float32)

    col_ids = jax.lax.broadcasted_iota(jnp.int32, (BLOCK, N), 1)
    row_vec = jax.lax.broadcasted_iota(jnp.int32, (BLOCK, 1), 0) + i * BLOCK
    diag = col_ids == row_vec
    neg = jnp.bfloat16(-3.0)
    c16 = jnp.where(diag, neg, (p * rncol_ref[...]).astype(jnp.bfloat16))

    match = lab_row_ref[...] == lab_mine
    zpack = jnp.where(match, p + 1024.0, p)

    acc = jnp.float32(0.0)
    for k in range(K):
        m = jnp.max(c16, axis=1, keepdims=True)
        is_max = c16 == m
        z = jnp.max(jnp.where(is_max, zpack, -1e9), axis=1,
                    keepdims=True)
        if k + 1 < K:
            c16 = jnp.where(is_max, neg, c16)
        t = jnp.floor(z * (1.0 / 1024.0) + 0.5)
        p_sel = z - t * 1024.0
        x = p_sel * n_rows
        bce = jnp.maximum(x, 0.0) - x * t + jnp.log1p(jnp.exp(-jnp.abs(x)))
        acc += jnp.sum(bce)

    out_ref[...] += (acc * (1.0 / (N * K))).reshape(1, 1)


def kernel(batch, labels):
    labels_f = labels.astype(jnp.float32)
    lab_row = labels_f.reshape(1, N)
    lab_col = labels_f.reshape(N, 1)
    out = pl.pallas_call(
        _loss_block_kernel,
        grid=(NBLK,),
        in_specs=[
            pl.BlockSpec((N, D), lambda i: (0, 0)),
            pl.BlockSpec((1, N), lambda i: (0, 0)),
            pl.BlockSpec((BLOCK, 1), lambda i: (i, 0)),
        ],
        out_specs=pl.BlockSpec((1, 1), lambda i: (0, 0)),
        out_shape=jax.ShapeDtypeStruct((1, 1), jnp.float32),
        scratch_shapes=[
            pltpu.VMEM((1, N), jnp.float32),
        ],
    )(batch, lab_row, lab_col)
    return out[0, 0]


# speedup vs baseline: 1.2282x; 1.2282x over previous
import jax
import jax.numpy as jnp
from jax.experimental import pallas as pl
from jax.experimental.pallas import tpu as pltpu

N = 4096
D = 1024
K = 5
BLOCK = 256
NBLK = N // BLOCK


def _loss_block_kernel(full_ref, lab_row_ref, lab_col_ref, out_ref,
                       fulln_ref, nrow_ref, gpack_ref):
    i = pl.program_id(0)
    lab_mine = lab_col_ref[...]

    @pl.when(i == 0)
    def _normalize():
        full = full_ref[...]
        sq = full * full
        row_ss = jnp.sum(sq, axis=1, keepdims=True)
        n_row = jnp.maximum(jnp.sqrt(row_ss), 1e-12)
        nrow_ref[...] = n_row
        fulln_ref[...] = full * (1.0 / n_row)
        ones_row = jnp.ones((1, D), dtype=jnp.float32)
        col_ss = jax.lax.dot_general(ones_row, sq, (((1,), (1,)), ((), ())),
                                     preferred_element_type=jnp.float32)
        n_col = jnp.maximum(jnp.sqrt(col_ss), 1e-12)
        gpack_ref[...] = lab_row_ref[...] * 256.0 + n_col

    rows_n = fulln_ref[pl.ds(i * BLOCK, BLOCK), :]
    n_rows = nrow_ref[pl.ds(i * BLOCK, BLOCK), :]
    gpack = gpack_ref[...]

    c = jax.lax.dot_general(rows_n, fulln_ref[...], (((1,), (1,)), ((), ())),
                            preferred_element_type=jnp.float32)

    col_ids = jax.lax.broadcasted_iota(jnp.int32, (BLOCK, N), 1)
    row_vec = jax.lax.broadcasted_iota(jnp.int32, (BLOCK, 1), 0) + i * BLOCK
    neg = jnp.float32(-3.0)
    c = jnp.where(col_ids == row_vec, neg, c)

    acc = jnp.float32(0.0)
    for k in range(K):
        m = jnp.max(c, axis=1, keepdims=True)
        is_max = c == m
        g = jnp.max(jnp.where(is_max, gpack, -1.0), axis=1,
                    keepdims=True)
        if k + 1 < K:
            c = jnp.where(is_max, neg, c)
        lab_j = jnp.floor(g * (1.0 / 256.0))
        n_j = g - lab_j * 256.0
        t = (lab_j == lab_mine).astype(jnp.float32)
        x = m * n_rows * n_j
        bce = jnp.maximum(x, 0.0) - x * t + jnp.log1p(jnp.exp(-jnp.abs(x)))
        acc += jnp.sum(bce)

    @pl.when(i == 0)
    def _init():
        out_ref[...] = jnp.zeros((1, 1), jnp.float32)

    out_ref[...] += (acc * (1.0 / (N * K))).reshape(1, 1)


def kernel(batch, labels):
    labels_f = labels.astype(jnp.float32)
    lab_row = labels_f.reshape(1, N)
    lab_col = labels_f.reshape(N, 1)
    out = pl.pallas_call(
        _loss_block_kernel,
        grid=(NBLK,),
        in_specs=[
            pl.BlockSpec((N, D), lambda i: (0, 0)),
            pl.BlockSpec((1, N), lambda i: (0, 0)),
            pl.BlockSpec((BLOCK, 1), lambda i: (i, 0)),
        ],
        out_specs=pl.BlockSpec((1, 1), lambda i: (0, 0)),
        out_shape=jax.ShapeDtypeStruct((1, 1), jnp.float32),
        scratch_shapes=[
            pltpu.VMEM((N, D), jnp.float32),
            pltpu.VMEM((N, 1), jnp.float32),
            pltpu.VMEM((1, N), jnp.float32),
        ],
    )(batch, lab_row, lab_col)
    return out[0, 0]


# speedup vs baseline: 1.3162x; 1.0716x over previous
import jax
import jax.numpy as jnp
from jax.experimental import pallas as pl
from jax.experimental.pallas import tpu as pltpu

N = 4096
D = 1024
K = 5
BLOCK = 1024
NBLK = N // BLOCK


def _loss_block_kernel(full_ref, lab_row_ref, lab_col_ref, out_ref,
                       fulln_ref, gpack_ref):
    i = pl.program_id(0)
    lab_mine = lab_col_ref[...]

    @pl.when(i == 0)
    def _normalize():
        full = full_ref[...]
        sq = full * full
        row_ss = jnp.sum(sq, axis=1, keepdims=True)
        n_row = jnp.maximum(jnp.sqrt(row_ss), 1e-12)
        fulln_ref[...] = full * (1.0 / n_row)
        ones_row = jnp.ones((1, D), dtype=jnp.float32)
        col_ss = jax.lax.dot_general(ones_row, sq, (((1,), (1,)), ((), ())),
                                     preferred_element_type=jnp.float32)
        n_col = jnp.maximum(jnp.sqrt(col_ss), 1e-12)
        gpack_ref[...] = lab_row_ref[...] * 256.0 + n_col

    rows_n = fulln_ref[pl.ds(i * BLOCK, BLOCK), :]
    rows_raw = full_ref[pl.ds(i * BLOCK, BLOCK), :]
    row_ss = jnp.sum(rows_raw * rows_raw, axis=1, keepdims=True)
    n_rows = jnp.maximum(jnp.sqrt(row_ss), 1e-12)
    gpack = gpack_ref[...]

    c = jax.lax.dot_general(rows_n, fulln_ref[...], (((1,), (1,)), ((), ())),
                            preferred_element_type=jnp.float32)

    col_ids = jax.lax.broadcasted_iota(jnp.int32, (BLOCK, N), 1)
    row_vec = jax.lax.broadcasted_iota(jnp.int32, (BLOCK, 1), 0) + i * BLOCK
    neg = jnp.float32(-3.0)
    c = jnp.where(col_ids == row_vec, neg, c)

    acc = jnp.float32(0.0)
    for k in range(K):
        m = jnp.max(c, axis=1, keepdims=True)
        is_max = c == m
        g = jnp.max(jnp.where(is_max, gpack, -1.0), axis=1,
                    keepdims=True)
        if k + 1 < K:
            c = jnp.where(is_max, neg, c)
        lab_j = jnp.floor(g * (1.0 / 256.0))
        n_j = g - lab_j * 256.0
        t = (lab_j == lab_mine).astype(jnp.float32)
        x = m * n_rows * n_j
        bce = jnp.maximum(x, 0.0) - x * t + jnp.log1p(jnp.exp(-jnp.abs(x)))
        acc += jnp.sum(bce)

    @pl.when(i == 0)
    def _init():
        out_ref[...] = jnp.zeros((1, 1), jnp.float32)

    out_ref[...] += (acc * (1.0 / (N * K))).reshape(1, 1)


def kernel(batch, labels):
    labels_f = labels.astype(jnp.float32)
    lab_row = labels_f.reshape(1, N)
    lab_col = labels_f.reshape(N, 1)
    out = pl.pallas_call(
        _loss_block_kernel,
        grid=(NBLK,),
        in_specs=[
            pl.BlockSpec((N, D), lambda i: (0, 0)),
            pl.BlockSpec((1, N), lambda i: (0, 0)),
            pl.BlockSpec((BLOCK, 1), lambda i: (i, 0)),
        ],
        out_specs=pl.BlockSpec((1, 1), lambda i: (0, 0)),
        out_shape=jax.ShapeDtypeStruct((1, 1), jnp.float32),
        scratch_shapes=[
            pltpu.VMEM((N, D), jnp.float32),
            pltpu.VMEM((1, N), jnp.float32),
        ],
    )(batch, lab_row, lab_col)
    return out[0, 0]
